# baseline (speedup 1.0000x reference)
import functools

import jax
import jax.numpy as jnp
from jax import lax
from jax.experimental import pallas as pl
from jax.experimental.pallas import tpu as pltpu
from jax.experimental.pallas import tpu_sc as plsc

VOCAB = 1000000
DIM = 32
BATCH = 16384
_ROWS_PER_BLOCK = 8
_BLOCK_W = _ROWS_PER_BLOCK * DIM
_NBLOCKS = VOCAB // _ROWS_PER_BLOCK

_info = plsc.get_sparse_core_info()
_NC, _NS, _L = _info.num_cores, _info.num_subcores, _info.num_lanes
_NW = _NC * _NS
_BPW = BATCH // _NW
_CHUNK = 64
_NCHUNK = _BPW // _CHUNK


def _make_gather():
    mesh = plsc.VectorSubcoreMesh(core_axis_name="c", subcore_axis_name="s")

    @functools.partial(
        pl.kernel,
        mesh=mesh,
        out_type=jax.ShapeDtypeStruct((BATCH, DIM), jnp.float32),
        scratch_types=[
            pltpu.VMEM((_BPW,), jnp.int32),
            pltpu.VMEM((_BPW,), jnp.int32),
            pltpu.VMEM((2, _CHUNK, _BLOCK_W), jnp.float32),
            pltpu.VMEM((_BPW, DIM), jnp.float32),
            pltpu.SemaphoreType.DMA,
            pltpu.SemaphoreType.DMA,
        ],
        compiler_params=pltpu.CompilerParams(needs_layout_passes=False),
    )
    def gather(table2_hbm, idx_hbm, out_hbm, idx_v, tid_v, blocks_v, rows_v,
               sem0, sem1):
        wid = lax.axis_index("s") * _NC + lax.axis_index("c")
        base = wid * _BPW
        pltpu.sync_copy(idx_hbm.at[pl.ds(base, _BPW)], idx_v)

        for g in range(_BPW // _L):
            v = idx_v[pl.ds(g * _L, _L)]
            tid_v[pl.ds(g * _L, _L)] = lax.shift_right_logical(v, 3)

        sems = [sem0, sem1]

        def fire(k, buf):
            return pltpu.async_copy(
                table2_hbm.at[tid_v.at[pl.ds(k * _CHUNK, _CHUNK)]],
                blocks_v.at[buf],
                sems[buf],
            )

        def select(k, buf):
            for g in range(_CHUNK // _L):
                j_vec = lax.iota(jnp.int32, _L) + g * _L
                iv = idx_v[pl.ds(k * _CHUNK + g * _L, _L)]
                off_vec = lax.shift_left(lax.bitwise_and(iv, 7), 5)
                for c in range(DIM):
                    c_vec = jnp.full((_L,), c, dtype=jnp.int32)
                    vals = plsc.load_gather(
                        blocks_v.at[buf], [j_vec, off_vec + c]
                    )
                    plsc.store_scatter(
                        rows_v, [j_vec + k * _CHUNK, c_vec], vals
                    )

        cps = [fire(0, 0), fire(1, 1)]
        for k in range(_NCHUNK):
            cps[k].wait()
            select(k, k % 2)
            if k + 2 < _NCHUNK:
                cps.append(fire(k + 2, k % 2))

        pltpu.sync_copy(rows_v, out_hbm.at[pl.ds(base, _BPW)])

    return gather


_gather = _make_gather()


@jax.jit
def kernel(warehouse_id, table):
    table2 = table.reshape(_NBLOCKS, _BLOCK_W)
    return _gather(table2, warehouse_id)


# speedup vs baseline: 2.9384x; 2.9384x over previous
import functools

import jax
import jax.numpy as jnp
from jax import lax
from jax.experimental import pallas as pl
from jax.experimental.pallas import tpu as pltpu
from jax.experimental.pallas import tpu_sc as plsc

VOCAB = 1000000
DIM = 32
BATCH = 16384
_ROWS_PER_TILE = 8
_NTILES = VOCAB // _ROWS_PER_TILE

_info = plsc.get_sparse_core_info()
_NC, _NS, _L = _info.num_cores, _info.num_subcores, _info.num_lanes
_NW = _NC * _NS
_BPW = BATCH // _NW


def _make_gather():
    mesh = plsc.VectorSubcoreMesh(core_axis_name="c", subcore_axis_name="s")

    @functools.partial(
        pl.kernel,
        mesh=mesh,
        out_type=jax.ShapeDtypeStruct((BATCH, DIM), jnp.float32),
        scratch_types=[
            pltpu.VMEM((_BPW,), jnp.int32),
            pltpu.VMEM((_BPW, DIM), jnp.float32),
            pltpu.SemaphoreType.DMA,
        ],
        compiler_params=pltpu.CompilerParams(needs_layout_passes=False),
    )
    def gather(table3_hbm, idx_hbm, out_hbm, idx_v, rows_v, sem):
        wid = lax.axis_index("s") * _NC + lax.axis_index("c")
        base = wid * _BPW
        pltpu.sync_copy(idx_hbm.at[pl.ds(base, _BPW)], idx_v)

        def body(g, carry):
            iv = idx_v[pl.ds(g * _L, _L)]
            for l in range(_L):
                ix = iv[l]
                t = lax.shift_right_logical(ix, 3)
                r = lax.bitwise_and(ix, 7)
                pltpu.async_copy(table3_hbm.at[t, r], rows_v.at[g * _L + l], sem)
            return carry

        lax.fori_loop(0, _BPW // _L, body, 0)
        pltpu.make_async_copy(out_hbm.at[pl.ds(base, _BPW)], rows_v, sem).wait()
        pltpu.sync_copy(rows_v, out_hbm.at[pl.ds(base, _BPW)])

    return gather


_gather = _make_gather()


@jax.jit
def kernel(warehouse_id, table):
    table3 = table.reshape(_NTILES, _ROWS_PER_TILE, DIM)
    return _gather(table3, warehouse_id)
